# baseline (speedup 1.0000x reference)
import functools

import jax
import jax.numpy as jnp
from jax.experimental import pallas as pl
from jax.experimental.pallas import tpu as pltpu

_B = 32768
_D_DOR = 128
_D_VEN = 128
_N_EXP = 8
_D_EXP = 64
_D_OUT = 64
_FB = 0.5


def _moe_kernel(d_ref, v_ref, gwd_ref, gwv_ref, w1d_ref, w1v_ref, b1_ref,
                w2o_ref, b2o_ref, bo_ref, wf_ref, exp_ref, tri_ref,
                assoc_ref, fbd_ref, fbv_ref, gw_ref):
    d = d_ref[...]
    v = v_ref[...]
    f32 = jnp.float32

    logits = (jnp.dot(d, gwd_ref[...], preferred_element_type=f32)
              + jnp.dot(v, gwv_ref[...], preferred_element_type=f32))

    tri = tri_ref[...]
    m1 = jnp.max(logits, axis=-1, keepdims=True)
    eq1 = (logits == m1).astype(f32)
    c1 = jnp.dot(eq1, tri, preferred_element_type=f32)
    first1 = eq1 * (c1 == 1.0)
    l2 = jnp.where(first1 > 0.0, jnp.float32(-1e30), logits)
    m2 = jnp.max(l2, axis=-1, keepdims=True)
    eq2 = (l2 == m2).astype(f32)
    c2 = jnp.dot(eq2, tri, preferred_element_type=f32)
    keep = (first1 + eq2 * (c2 == 1.0)) > 0.0

    e = jnp.where(keep, jnp.exp(logits - m1), 0.0)
    gw = e / jnp.sum(e, axis=-1, keepdims=True)
    gw_ref[...] = gw

    bf16 = jnp.bfloat16
    d16 = d.astype(bf16)
    v16 = v.astype(bf16)
    h = (jnp.dot(d16, w1d_ref[...], preferred_element_type=f32)
         + jnp.dot(v16, w1v_ref[...], preferred_element_type=f32)
         + b1_ref[...])
    h = 0.5 * h * (1.0 + jax.lax.erf(h * jnp.float32(0.7071067811865476)))

    gwx = jnp.dot(gw, exp_ref[...], preferred_element_type=f32)
    hs = (h * gwx).astype(bf16)
    assoc = (jnp.dot(hs, w2o_ref[...], preferred_element_type=f32)
             + jnp.dot(gw, b2o_ref[...], preferred_element_type=f32)
             + bo_ref[...])
    assoc_ref[...] = assoc
    fb = _FB * jnp.dot(assoc.astype(bf16), wf_ref[...], preferred_element_type=f32)
    fbd_ref[...] = fb[:, :_D_DOR]
    fbv_ref[...] = fb[:, _D_DOR:]


@functools.partial(jax.jit, static_argnames=("block",))
def _run(dorsal, ventral, gate_w, w1, b1, w2, b2, wo, bo, wfd, wfv, block=4096):
    gwT = gate_w.T
    gwd, gwv = gwT[:_D_DOR], gwT[_D_DOR:]
    w1cat = w1.transpose(2, 0, 1).reshape(_D_DOR + _D_VEN, _N_EXP * _D_EXP)
    w1cat = w1cat.astype(jnp.bfloat16)
    w1d, w1v = w1cat[:_D_DOR], w1cat[_D_DOR:]
    b1row = b1.reshape(1, _N_EXP * _D_EXP)
    w2s = w2.transpose(0, 2, 1).reshape(_N_EXP * _D_EXP, _D_EXP)
    w2o = (w2s @ wo.T).astype(jnp.bfloat16)
    b2o = b2 @ wo.T
    borow = bo.reshape(1, _D_OUT)
    wf = jnp.concatenate([wfd.T, wfv.T], axis=1).astype(jnp.bfloat16)
    expand = jnp.repeat(jnp.eye(_N_EXP, dtype=jnp.float32), _D_EXP, axis=1)
    tri = jnp.triu(jnp.ones((_N_EXP, _N_EXP), dtype=jnp.float32))

    grid = (_B // block,)
    tok = lambda i: (i, 0)
    full = lambda i: (0, 0)
    out_shapes = (
        jax.ShapeDtypeStruct((_B, _D_OUT), jnp.float32),
        jax.ShapeDtypeStruct((_B, _D_DOR), jnp.float32),
        jax.ShapeDtypeStruct((_B, _D_VEN), jnp.float32),
        jax.ShapeDtypeStruct((_B, _N_EXP), jnp.float32),
    )
    return pl.pallas_call(
        _moe_kernel,
        grid=grid,
        in_specs=[
            pl.BlockSpec((block, _D_DOR), tok),
            pl.BlockSpec((block, _D_VEN), tok),
            pl.BlockSpec((_D_DOR, _N_EXP), full),
            pl.BlockSpec((_D_VEN, _N_EXP), full),
            pl.BlockSpec((_D_DOR, _N_EXP * _D_EXP), full),
            pl.BlockSpec((_D_VEN, _N_EXP * _D_EXP), full),
            pl.BlockSpec((1, _N_EXP * _D_EXP), full),
            pl.BlockSpec((_N_EXP * _D_EXP, _D_OUT), full),
            pl.BlockSpec((_N_EXP, _D_OUT), full),
            pl.BlockSpec((1, _D_OUT), full),
            pl.BlockSpec((_D_EXP, _D_DOR + _D_VEN), full),
            pl.BlockSpec((_N_EXP, _N_EXP * _D_EXP), full),
            pl.BlockSpec((_N_EXP, _N_EXP), full),
        ],
        out_specs=(
            pl.BlockSpec((block, _D_OUT), tok),
            pl.BlockSpec((block, _D_DOR), tok),
            pl.BlockSpec((block, _D_VEN), tok),
            pl.BlockSpec((block, _N_EXP), tok),
        ),
        out_shape=out_shapes,
        compiler_params=pltpu.CompilerParams(
            dimension_semantics=("parallel",),
        ),
    )(dorsal, ventral, gwd, gwv, w1d, w1v, b1row, w2o, b2o, borow, wf, expand,
      tri)


def kernel(dorsal, ventral, gate_w, w1, b1, w2, b2, wo, bo, wfd, wfv):
    return _run(dorsal, ventral, gate_w, w1, b1, w2, b2, wo, bo, wfd, wfv)


# speedup vs baseline: 1.0143x; 1.0143x over previous
import functools

import jax
import jax.numpy as jnp
from jax.experimental import pallas as pl
from jax.experimental.pallas import tpu as pltpu

_B = 32768
_D_DOR = 128
_D_VEN = 128
_N_EXP = 8
_D_EXP = 64
_D_OUT = 64
_FB = 0.5


def _moe_kernel(d_ref, v_ref, gwd_ref, gwv_ref, w1d_ref, w1v_ref, b1_ref,
                w2o_ref, b2o_ref, bo_ref, wf_ref, exp_ref, tri_ref,
                assoc_ref, fbd_ref, fbv_ref, gw_ref):
    d = d_ref[...]
    v = v_ref[...]
    f32 = jnp.float32

    logits = (jnp.dot(d, gwd_ref[...], preferred_element_type=f32)
              + jnp.dot(v, gwv_ref[...], preferred_element_type=f32))

    tri = tri_ref[...]
    m1 = jnp.max(logits, axis=-1, keepdims=True)
    eq1 = (logits == m1).astype(f32)
    c1 = jnp.dot(eq1, tri, preferred_element_type=f32)
    first1 = eq1 * (c1 == 1.0)
    l2 = jnp.where(first1 > 0.0, jnp.float32(-1e30), logits)
    m2 = jnp.max(l2, axis=-1, keepdims=True)
    eq2 = (l2 == m2).astype(f32)
    c2 = jnp.dot(eq2, tri, preferred_element_type=f32)
    keep = (first1 + eq2 * (c2 == 1.0)) > 0.0

    e = jnp.where(keep, jnp.exp(logits - m1), 0.0)
    gw = e / jnp.sum(e, axis=-1, keepdims=True)
    gw_ref[...] = gw

    bf16 = jnp.bfloat16
    d16 = d.astype(bf16)
    v16 = v.astype(bf16)
    h = (jnp.dot(d16, w1d_ref[...], preferred_element_type=f32)
         + jnp.dot(v16, w1v_ref[...], preferred_element_type=f32)
         + b1_ref[...])
    h = 0.5 * h * (1.0 + jax.lax.erf(h * jnp.float32(0.7071067811865476)))

    gwx = jnp.dot(gw, exp_ref[...], preferred_element_type=f32)
    hs = (h * gwx).astype(bf16)
    assoc = (jnp.dot(hs, w2o_ref[...], preferred_element_type=f32)
             + jnp.dot(gw, b2o_ref[...], preferred_element_type=f32)
             + bo_ref[...])
    assoc_ref[...] = assoc
    fb = _FB * jnp.dot(assoc.astype(bf16), wf_ref[...], preferred_element_type=f32)
    fbd_ref[...] = fb[:, :_D_DOR]
    fbv_ref[...] = fb[:, _D_DOR:]


@functools.partial(jax.jit, static_argnames=("block",))
def _run(dorsal, ventral, gate_w, w1, b1, w2, b2, wo, bo, wfd, wfv, block=2048):
    gwT = gate_w.T
    gwd, gwv = gwT[:_D_DOR], gwT[_D_DOR:]
    w1cat = w1.transpose(2, 0, 1).reshape(_D_DOR + _D_VEN, _N_EXP * _D_EXP)
    w1cat = w1cat.astype(jnp.bfloat16)
    w1d, w1v = w1cat[:_D_DOR], w1cat[_D_DOR:]
    b1row = b1.reshape(1, _N_EXP * _D_EXP)
    w2s = w2.transpose(0, 2, 1).reshape(_N_EXP * _D_EXP, _D_EXP)
    w2o = (w2s @ wo.T).astype(jnp.bfloat16)
    b2o = b2 @ wo.T
    borow = bo.reshape(1, _D_OUT)
    wf = jnp.concatenate([wfd.T, wfv.T], axis=1).astype(jnp.bfloat16)
    expand = jnp.repeat(jnp.eye(_N_EXP, dtype=jnp.float32), _D_EXP, axis=1)
    tri = jnp.triu(jnp.ones((_N_EXP, _N_EXP), dtype=jnp.float32))

    grid = (_B // block,)
    tok = lambda i: (i, 0)
    full = lambda i: (0, 0)
    out_shapes = (
        jax.ShapeDtypeStruct((_B, _D_OUT), jnp.float32),
        jax.ShapeDtypeStruct((_B, _D_DOR), jnp.float32),
        jax.ShapeDtypeStruct((_B, _D_VEN), jnp.float32),
        jax.ShapeDtypeStruct((_B, _N_EXP), jnp.float32),
    )
    return pl.pallas_call(
        _moe_kernel,
        grid=grid,
        in_specs=[
            pl.BlockSpec((block, _D_DOR), tok),
            pl.BlockSpec((block, _D_VEN), tok),
            pl.BlockSpec((_D_DOR, _N_EXP), full),
            pl.BlockSpec((_D_VEN, _N_EXP), full),
            pl.BlockSpec((_D_DOR, _N_EXP * _D_EXP), full),
            pl.BlockSpec((_D_VEN, _N_EXP * _D_EXP), full),
            pl.BlockSpec((1, _N_EXP * _D_EXP), full),
            pl.BlockSpec((_N_EXP * _D_EXP, _D_OUT), full),
            pl.BlockSpec((_N_EXP, _D_OUT), full),
            pl.BlockSpec((1, _D_OUT), full),
            pl.BlockSpec((_D_EXP, _D_DOR + _D_VEN), full),
            pl.BlockSpec((_N_EXP, _N_EXP * _D_EXP), full),
            pl.BlockSpec((_N_EXP, _N_EXP), full),
        ],
        out_specs=(
            pl.BlockSpec((block, _D_OUT), tok),
            pl.BlockSpec((block, _D_DOR), tok),
            pl.BlockSpec((block, _D_VEN), tok),
            pl.BlockSpec((block, _N_EXP), tok),
        ),
        out_shape=out_shapes,
        compiler_params=pltpu.CompilerParams(
            dimension_semantics=("parallel",),
        ),
    )(dorsal, ventral, gwd, gwv, w1d, w1v, b1row, w2o, b2o, borow, wf, expand,
      tri)


def kernel(dorsal, ventral, gate_w, w1, b1, w2, b2, wo, bo, wfd, wfv):
    return _run(dorsal, ventral, gate_w, w1, b1, w2, b2, wo, bo, wfd, wfv)


# speedup vs baseline: 1.1345x; 1.1185x over previous
import functools

import jax
import jax.numpy as jnp
from jax.experimental import pallas as pl
from jax.experimental.pallas import tpu as pltpu

_B = 32768
_D_DOR = 128
_D_VEN = 128
_N_EXP = 8
_D_EXP = 64
_D_OUT = 64
_FB = 0.5


def _moe_kernel(d_ref, v_ref, gwd_ref, gwv_ref, w1d_ref, w1v_ref, b1_ref,
                w2o_ref, b2o_ref, bo_ref, wf_ref, exp_ref, tri_ref,
                assoc_ref, fbd_ref, fbv_ref, gw_ref):
    d = d_ref[...]
    v = v_ref[...]
    f32 = jnp.float32
    contract0 = (((0,), (0,)), ((), ()))
    contract11 = (((1,), (1,)), ((), ()))

    lt = (jax.lax.dot_general(gwd_ref[...], d, contract11,
                              preferred_element_type=f32)
          + jax.lax.dot_general(gwv_ref[...], v, contract11,
                                preferred_element_type=f32))

    tri = tri_ref[...]
    m1 = jnp.max(lt, axis=0, keepdims=True)
    eq1 = (lt == m1).astype(f32)
    c1 = jnp.dot(tri, eq1, preferred_element_type=f32)
    first1 = eq1 * (c1 == 1.0)
    l2 = jnp.where(first1 > 0.0, jnp.float32(-1e30), lt)
    m2 = jnp.max(l2, axis=0, keepdims=True)
    eq2 = (l2 == m2).astype(f32)
    c2 = jnp.dot(tri, eq2, preferred_element_type=f32)
    keep = (first1 + eq2 * (c2 == 1.0)) > 0.0

    e = jnp.where(keep, jnp.exp(lt - m1), 0.0)
    gwt = e / jnp.sum(e, axis=0, keepdims=True)
    gw_ref[...] = gwt.T

    bf16 = jnp.bfloat16
    d16 = d.astype(bf16)
    v16 = v.astype(bf16)
    h = (jnp.dot(d16, w1d_ref[...], preferred_element_type=f32)
         + jnp.dot(v16, w1v_ref[...], preferred_element_type=f32)
         + b1_ref[...])
    h = h * (1.0 + jax.lax.erf(h * jnp.float32(0.7071067811865476)))

    gwx = jax.lax.dot_general(gwt, exp_ref[...], contract0,
                              preferred_element_type=f32)
    hs = (h * gwx).astype(bf16)
    assoc = (jnp.dot(hs, w2o_ref[...], preferred_element_type=f32)
             + jax.lax.dot_general(gwt, b2o_ref[...], contract0,
                                   preferred_element_type=f32)
             + bo_ref[...])
    assoc_ref[...] = assoc
    fb = _FB * jnp.dot(assoc.astype(bf16), wf_ref[...], preferred_element_type=f32)
    fbd_ref[...] = fb[:, :_D_DOR]
    fbv_ref[...] = fb[:, _D_DOR:]


@functools.partial(jax.jit, static_argnames=("block",))
def _run(dorsal, ventral, gate_w, w1, b1, w2, b2, wo, bo, wfd, wfv, block=4096):
    gwd, gwv = gate_w[:, :_D_DOR], gate_w[:, _D_DOR:]
    w1cat = w1.transpose(2, 0, 1).reshape(_D_DOR + _D_VEN, _N_EXP * _D_EXP)
    w1cat = w1cat.astype(jnp.bfloat16)
    w1d, w1v = w1cat[:_D_DOR], w1cat[_D_DOR:]
    b1row = b1.reshape(1, _N_EXP * _D_EXP)
    w2s = w2.transpose(0, 2, 1).reshape(_N_EXP * _D_EXP, _D_EXP)
    w2o = (0.5 * (w2s @ wo.T)).astype(jnp.bfloat16)
    b2o = b2 @ wo.T
    borow = bo.reshape(1, _D_OUT)
    wf = jnp.concatenate([wfd.T, wfv.T], axis=1).astype(jnp.bfloat16)
    expand = jnp.repeat(jnp.eye(_N_EXP, dtype=jnp.float32), _D_EXP, axis=1)
    tri = jnp.tril(jnp.ones((_N_EXP, _N_EXP), dtype=jnp.float32))

    grid = (_B // block,)
    tok = lambda i: (i, 0)
    full = lambda i: (0, 0)
    out_shapes = (
        jax.ShapeDtypeStruct((_B, _D_OUT), jnp.float32),
        jax.ShapeDtypeStruct((_B, _D_DOR), jnp.float32),
        jax.ShapeDtypeStruct((_B, _D_VEN), jnp.float32),
        jax.ShapeDtypeStruct((_B, _N_EXP), jnp.float32),
    )
    return pl.pallas_call(
        _moe_kernel,
        grid=grid,
        in_specs=[
            pl.BlockSpec((block, _D_DOR), tok),
            pl.BlockSpec((block, _D_VEN), tok),
            pl.BlockSpec((_N_EXP, _D_DOR), full),
            pl.BlockSpec((_N_EXP, _D_VEN), full),
            pl.BlockSpec((_D_DOR, _N_EXP * _D_EXP), full),
            pl.BlockSpec((_D_VEN, _N_EXP * _D_EXP), full),
            pl.BlockSpec((1, _N_EXP * _D_EXP), full),
            pl.BlockSpec((_N_EXP * _D_EXP, _D_OUT), full),
            pl.BlockSpec((_N_EXP, _D_OUT), full),
            pl.BlockSpec((1, _D_OUT), full),
            pl.BlockSpec((_D_EXP, _D_DOR + _D_VEN), full),
            pl.BlockSpec((_N_EXP, _N_EXP * _D_EXP), full),
            pl.BlockSpec((_N_EXP, _N_EXP), full),
        ],
        out_specs=(
            pl.BlockSpec((block, _D_OUT), tok),
            pl.BlockSpec((block, _D_DOR), tok),
            pl.BlockSpec((block, _D_VEN), tok),
            pl.BlockSpec((block, _N_EXP), tok),
        ),
        out_shape=out_shapes,
        compiler_params=pltpu.CompilerParams(
            dimension_semantics=("parallel",),
        ),
    )(dorsal, ventral, gwd, gwv, w1d, w1v, b1row, w2o, b2o, borow, wf, expand,
      tri)


def kernel(dorsal, ventral, gate_w, w1, b1, w2, b2, wo, bo, wfd, wfv):
    return _run(dorsal, ventral, gate_w, w1, b1, w2, b2, wo, bo, wfd, wfv)


# speedup vs baseline: 1.3081x; 1.1531x over previous
import functools

import jax
import jax.numpy as jnp
from jax.experimental import pallas as pl
from jax.experimental.pallas import tpu as pltpu

_B = 32768
_D_DOR = 128
_D_VEN = 128
_N_EXP = 8
_D_EXP = 64
_D_OUT = 64
_FB = 0.5


def _moe_kernel(d_ref, v_ref, gwd_ref, gwv_ref, w1d_ref, w1v_ref, b1_ref,
                w2o_ref, b2o_ref, bo_ref, wf_ref, exp_ref, tri_ref,
                assoc_ref, fbd_ref, fbv_ref, gw_ref):
    d = d_ref[...]
    v = v_ref[...]
    f32 = jnp.float32
    contract0 = (((0,), (0,)), ((), ()))
    contract11 = (((1,), (1,)), ((), ()))

    lt = (jax.lax.dot_general(gwd_ref[...], d, contract11,
                              preferred_element_type=f32)
          + jax.lax.dot_general(gwv_ref[...], v, contract11,
                                preferred_element_type=f32))

    tri = tri_ref[...]
    m1 = jnp.max(lt, axis=0, keepdims=True)
    eq1 = (lt == m1).astype(f32)
    c1 = jnp.dot(tri, eq1, preferred_element_type=f32)
    first1 = eq1 * (c1 == 1.0)
    l2 = jnp.where(first1 > 0.0, jnp.float32(-1e30), lt)
    m2 = jnp.max(l2, axis=0, keepdims=True)
    eq2 = (l2 == m2).astype(f32)
    c2 = jnp.dot(tri, eq2, preferred_element_type=f32)
    keep = (first1 + eq2 * (c2 == 1.0)) > 0.0

    e = jnp.where(keep, jnp.exp(lt - m1), 0.0)
    gwt = e / jnp.sum(e, axis=0, keepdims=True)
    gw_ref[...] = gwt

    bf16 = jnp.bfloat16
    n_sub = 2
    rows = d.shape[0] // n_sub
    for s in range(n_sub):
        sl = slice(s * rows, (s + 1) * rows)
        d16 = d[sl].astype(bf16)
        v16 = v[sl].astype(bf16)
        h = (jnp.dot(d16, w1d_ref[...], preferred_element_type=f32)
             + jnp.dot(v16, w1v_ref[...], preferred_element_type=f32)
             + b1_ref[...])
        h = h * (1.0 + jax.lax.erf(h * jnp.float32(0.7071067811865476)))

        gws = gwt[:, sl]
        gwx = jax.lax.dot_general(gws, exp_ref[...], contract0,
                                  preferred_element_type=f32)
        hs = (h * gwx).astype(bf16)
        assoc = (jnp.dot(hs, w2o_ref[...], preferred_element_type=f32)
                 + jax.lax.dot_general(gws, b2o_ref[...], contract0,
                                       preferred_element_type=f32)
                 + bo_ref[...])
        assoc_ref[sl] = assoc
        fb = _FB * jnp.dot(assoc.astype(bf16), wf_ref[...],
                           preferred_element_type=f32)
        fbd_ref[sl] = fb[:, :_D_DOR]
        fbv_ref[sl] = fb[:, _D_DOR:]


@functools.partial(jax.jit, static_argnames=("block",))
def _run(dorsal, ventral, gate_w, w1, b1, w2, b2, wo, bo, wfd, wfv, block=8192):
    gwd, gwv = gate_w[:, :_D_DOR], gate_w[:, _D_DOR:]
    w1cat = w1.transpose(2, 0, 1).reshape(_D_DOR + _D_VEN, _N_EXP * _D_EXP)
    w1cat = w1cat.astype(jnp.bfloat16)
    w1d, w1v = w1cat[:_D_DOR], w1cat[_D_DOR:]
    b1row = b1.reshape(1, _N_EXP * _D_EXP)
    w2s = w2.transpose(0, 2, 1).reshape(_N_EXP * _D_EXP, _D_EXP)
    w2o = (0.5 * (w2s @ wo.T)).astype(jnp.bfloat16)
    b2o = b2 @ wo.T
    borow = bo.reshape(1, _D_OUT)
    wf = jnp.concatenate([wfd.T, wfv.T], axis=1).astype(jnp.bfloat16)
    expand = jnp.repeat(jnp.eye(_N_EXP, dtype=jnp.float32), _D_EXP, axis=1)
    tri = jnp.tril(jnp.ones((_N_EXP, _N_EXP), dtype=jnp.float32))

    grid = (_B // block,)
    tok = lambda i: (i, 0)
    full = lambda i: (0, 0)
    out_shapes = (
        jax.ShapeDtypeStruct((_B, _D_OUT), jnp.float32),
        jax.ShapeDtypeStruct((_B, _D_DOR), jnp.float32),
        jax.ShapeDtypeStruct((_B, _D_VEN), jnp.float32),
        jax.ShapeDtypeStruct((_N_EXP, _B), jnp.float32),
    )
    assoc, fbd, fbv, gwt = pl.pallas_call(
        _moe_kernel,
        grid=grid,
        in_specs=[
            pl.BlockSpec((block, _D_DOR), tok),
            pl.BlockSpec((block, _D_VEN), tok),
            pl.BlockSpec((_N_EXP, _D_DOR), full),
            pl.BlockSpec((_N_EXP, _D_VEN), full),
            pl.BlockSpec((_D_DOR, _N_EXP * _D_EXP), full),
            pl.BlockSpec((_D_VEN, _N_EXP * _D_EXP), full),
            pl.BlockSpec((1, _N_EXP * _D_EXP), full),
            pl.BlockSpec((_N_EXP * _D_EXP, _D_OUT), full),
            pl.BlockSpec((_N_EXP, _D_OUT), full),
            pl.BlockSpec((1, _D_OUT), full),
            pl.BlockSpec((_D_EXP, _D_DOR + _D_VEN), full),
            pl.BlockSpec((_N_EXP, _N_EXP * _D_EXP), full),
            pl.BlockSpec((_N_EXP, _N_EXP), full),
        ],
        out_specs=(
            pl.BlockSpec((block, _D_OUT), tok),
            pl.BlockSpec((block, _D_DOR), tok),
            pl.BlockSpec((block, _D_VEN), tok),
            pl.BlockSpec((_N_EXP, block), lambda i: (0, i)),
        ),
        out_shape=out_shapes,
        compiler_params=pltpu.CompilerParams(
            dimension_semantics=("parallel",),
        ),
    )(dorsal, ventral, gwd, gwv, w1d, w1v, b1row, w2o, b2o, borow, wf, expand,
      tri)
    return assoc, fbd, fbv, gwt.T


def kernel(dorsal, ventral, gate_w, w1, b1, w2, b2, wo, bo, wfd, wfv):
    return _run(dorsal, ventral, gate_w, w1, b1, w2, b2, wo, bo, wfd, wfv)
